# baseline (speedup 1.0000x reference)
import functools

import jax
import jax.numpy as jnp
from jax import lax
from jax.experimental import pallas as pl
from jax.experimental.pallas import tpu as pltpu
from jax.experimental.pallas import tpu_sc as plsc

NUM_CORES = 2
NUM_SUBCORES = 16
NW = NUM_CORES * NUM_SUBCORES
LANES = 16
DB_ROWS = 16


@functools.lru_cache(maxsize=None)
def _build(N, D, V):
    VW = ((V + NW - 1) // NW + 7) // 8 * 8
    NBLK = D // DB_ROWS
    BKT = N + LANES

    mesh = plsc.VectorSubcoreMesh(
        core_axis_name="c", subcore_axis_name="s",
        num_cores=NUM_CORES, num_subcores=NUM_SUBCORES,
    )

    @functools.partial(
        pl.kernel,
        out_type=jax.ShapeDtypeStruct((N * NBLK, DB_ROWS), jnp.float32),
        mesh=mesh,
        compiler_params=pltpu.CompilerParams(use_tc_tiling_on_sc=False, needs_layout_passes=False),
        scratch_types=[
            pltpu.VMEM((N,), jnp.int32),
            pltpu.VMEM((BKT,), jnp.int32),
            pltpu.VMEM((BKT,), jnp.int32),
            pltpu.VMEM((DB_ROWS, VW), jnp.float32),
            pltpu.VMEM((128, DB_ROWS), jnp.float32),
            pltpu.VMEM((128,), jnp.int32),
        ],
    )
    def run(x_hbm, w_hbm, out_hbm, x_v, bkt_v, bkt_p, chunk, staging, sidx):
        c = lax.axis_index("c")
        s = lax.axis_index("s")
        wid = s * NUM_CORES + c
        lo = wid * VW
        hi = lo + VW
        lo_dma = jnp.minimum(lo, V - VW)
        iota = lax.iota(jnp.int32, LANES)

        pltpu.sync_copy(x_hbm, x_v)

        def bucket_body(g, cnt):
            xv = x_v[pl.ds(g * LANES, LANES)]
            m = (xv >= lo) & (xv < hi)
            mi = m.astype(jnp.int32)
            incl = plsc.cumsum(mi)
            dst = cnt + incl - mi
            plsc.store_scatter(bkt_v, [dst], xv - lo_dma, mask=m)
            plsc.store_scatter(bkt_p, [dst], g * LANES + iota, mask=m)
            return cnt + jnp.sum(mi)

        cnt = lax.fori_loop(0, N // LANES, bucket_body, jnp.int32(0))
        nslab = (cnt + 127) >> 7

        def dblk_body(db, _):
            pltpu.sync_copy(w_hbm.at[pl.ds(db * DB_ROWS, DB_ROWS), pl.ds(lo_dma, VW)], chunk)

            def slab_body(sl, _2):
                base = sl * 128
                for g in range(8):
                    jv = base + g * LANES + iota
                    jc = jnp.minimum(jv, cnt - 1)
                    vs = plsc.load_gather(bkt_v, [jc])
                    ps = plsc.load_gather(bkt_p, [jc])
                    sidx[pl.ds(g * LANES, LANES)] = (ps << 6) + db if NBLK == 64 else ps * NBLK + db
                    rowv = g * LANES + iota
                    for d in range(DB_ROWS):
                        dv = jnp.full((LANES,), d, jnp.int32)
                        col = plsc.load_gather(chunk, [dv, vs])
                        plsc.store_scatter(staging, [rowv, dv], col)
                pltpu.sync_copy(staging, out_hbm.at[sidx])
                return 0

            lax.fori_loop(0, nslab, slab_body, 0)
            return 0

        lax.fori_loop(0, NBLK, dblk_body, 0)

    return run


def kernel(x, W_E):
    B, S = x.shape
    D, V = W_E.shape
    N = B * S
    x_flat = x.reshape(N).astype(jnp.int32)
    out = _build(N, D, V)(x_flat, W_E)
    return out.reshape(B, S, D)


# speedup vs baseline: 1.0008x; 1.0008x over previous
import functools

import jax
import jax.numpy as jnp
from jax import lax
from jax.experimental import pallas as pl
from jax.experimental.pallas import tpu as pltpu
from jax.experimental.pallas import tpu_sc as plsc

NUM_CORES = 2
NUM_SUBCORES = 16
NW = NUM_CORES * NUM_SUBCORES
LANES = 16
CHUNK = 16


@functools.lru_cache(maxsize=None)
def _build(N, D, V):
    NI = N // NW
    NCH = NI // CHUNK
    CW = CHUNK * D
    NDG = D // LANES

    mesh = plsc.VectorSubcoreMesh(
        core_axis_name="c", subcore_axis_name="s",
        num_cores=NUM_CORES, num_subcores=NUM_SUBCORES,
    )

    @functools.partial(
        pl.kernel,
        out_type=jax.ShapeDtypeStruct((N * D,), jnp.float32),
        mesh=mesh,
        compiler_params=pltpu.CompilerParams(
            use_tc_tiling_on_sc=False, needs_layout_passes=False),
        scratch_types=[
            pltpu.VMEM((NI,), jnp.int32),
            pltpu.VMEM((CW,), jnp.int32),
            pltpu.VMEM((CW,), jnp.int32),
            pltpu.VMEM((CW,), jnp.float32),
            pltpu.VMEM((CW,), jnp.float32),
            pltpu.SemaphoreType.DMA,
            pltpu.SemaphoreType.DMA,
            pltpu.SemaphoreType.DMA,
            pltpu.SemaphoreType.DMA,
        ],
    )
    def run(x_hbm, w_hbm, out_hbm, xl_v, idxA, idxB, dstA, dstB,
            sgA, sgB, soA, soB):
        c = lax.axis_index("c")
        s = lax.axis_index("s")
        wid = s * NUM_CORES + c
        i0 = wid * NI
        iota = lax.iota(jnp.int32, LANES)
        iotaV = iota * V

        pltpu.sync_copy(x_hbm.at[pl.ds(i0, NI)], xl_v)

        def build_idx(idx_ref, ch):
            for j in range(CHUNK):
                xj = plsc.load_gather(
                    xl_v, [jnp.full((LANES,), ch * CHUNK + j, jnp.int32)])
                v = xj + iotaV
                idx_ref[pl.ds(j * D, LANES)] = v
                for g in range(1, NDG):
                    v = v + LANES * V
                    idx_ref[pl.ds(j * D + g * LANES, LANES)] = v

        def start_gather(idx_ref, dst_ref, sem):
            pltpu.async_copy(w_hbm.at[idx_ref], dst_ref, sem)

        def wait_gather(idx_ref, dst_ref, sem):
            pltpu.make_async_copy(w_hbm.at[idx_ref], dst_ref, sem).wait()

        def out_slice(ch):
            return out_hbm.at[pl.ds((i0 + ch * CHUNK) * D, CW)]

        def start_out(dst_ref, ch, sem):
            pltpu.async_copy(dst_ref, out_slice(ch), sem)

        def wait_out(dst_ref, ch, sem):
            pltpu.make_async_copy(dst_ref, out_slice(ch), sem).wait()

        def body(ch, _):
            build_idx(idxA, ch)
            start_gather(idxA, dstA, sgA)
            wait_gather(idxA, dstA, sgA)
            start_out(dstA, ch, soA)
            wait_out(dstA, ch, soA)
            return 0

        lax.fori_loop(0, NCH, body, 0)

    return run


def kernel(x, W_E):
    B, S = x.shape
    D, V = W_E.shape
    N = B * S
    x_flat = x.reshape(N).astype(jnp.int32)
    w_flat = W_E.reshape(D * V)
    out = _build(N, D, V)(x_flat, w_flat)
    return out.reshape(B, S, D)


# speedup vs baseline: 1.0061x; 1.0052x over previous
import functools

import jax
import jax.numpy as jnp
from jax import lax
from jax.experimental import pallas as pl
from jax.experimental.pallas import tpu as pltpu
from jax.experimental.pallas import tpu_sc as plsc

NUM_CORES = 2
NUM_SUBCORES = 16
NW = NUM_CORES * NUM_SUBCORES
LANES = 16
CHUNK = 8


@functools.lru_cache(maxsize=None)
def _build(N, D, V):
    NI = N // NW
    NCH = NI // CHUNK
    CW = CHUNK * D
    NDG = D // LANES
    NDESC = CHUNK * NDG

    mesh = plsc.VectorSubcoreMesh(
        core_axis_name="c", subcore_axis_name="s",
        num_cores=NUM_CORES, num_subcores=NUM_SUBCORES,
    )

    @functools.partial(
        pl.kernel,
        out_type=jax.ShapeDtypeStruct((N * D,), jnp.float32),
        mesh=mesh,
        compiler_params=pltpu.CompilerParams(
            use_tc_tiling_on_sc=False, needs_layout_passes=False),
        scratch_types=[
            pltpu.VMEM((NI,), jnp.int32),
            pltpu.VMEM((CW,), jnp.float32),
            pltpu.VMEM((CW,), jnp.float32),
            pltpu.VMEM((CW,), jnp.int32),
            pltpu.SemaphoreType.DMA,
            pltpu.SemaphoreType.DMA,
            pltpu.SemaphoreType.DMA,
            pltpu.SemaphoreType.DMA,
        ],
    )
    def run(x_hbm, w_hbm, out_hbm, xl_v, dstA, dstB, dumm, sgA, sgB, soA, soB):
        c = lax.axis_index("c")
        s = lax.axis_index("s")
        wid = s * NUM_CORES + c
        i0 = wid * NI
        iota = lax.iota(jnp.int32, LANES)
        iotaV = iota * V
        zeros = jnp.zeros((LANES,), jnp.int32)

        pltpu.sync_copy(x_hbm.at[pl.ds(i0, NI)], xl_v)

        def issue_chunk(ch, dst_ref, sem):
            for j in range(CHUNK):
                xj = plsc.load_gather(
                    xl_v, [jnp.full((LANES,), ch * CHUNK + j, jnp.int32)])
                v = xj + iotaV
                for g in range(NDG):
                    pltpu.async_copy(
                        w_hbm.at[v],
                        dst_ref.at[pl.ds((j * NDG + g) * LANES, LANES)], sem)
                    v = v + LANES * V

        def drain_chunk(dst_ref, sem):
            pltpu.make_async_copy(w_hbm.at[dumm], dst_ref, sem).wait()

        def out_slice(ch):
            return out_hbm.at[pl.ds((i0 + ch * CHUNK) * D, CW)]

        def start_out(dst_ref, ch, sem):
            pltpu.async_copy(dst_ref, out_slice(ch), sem)

        def wait_out(dst_ref, ch, sem):
            pltpu.make_async_copy(dst_ref, out_slice(ch), sem).wait()

        def body(h, _):
            a = 2 * h
            b = a + 1
            issue_chunk(a, dstA, sgA)
            drain_chunk(dstA, sgA)

            @pl.when(h > 0)
            def _():
                wait_out(dstB, b - 2, soB)
            start_out(dstA, a, soA)

            issue_chunk(b, dstB, sgB)
            drain_chunk(dstB, sgB)
            wait_out(dstA, a, soA)
            start_out(dstB, b, soB)
            return 0

        lax.fori_loop(0, NCH // 2, body, 0)
        wait_out(dstB, NCH - 1, soB)

    return run


def kernel(x, W_E):
    B, S = x.shape
    D, V = W_E.shape
    N = B * S
    x_flat = x.reshape(N).astype(jnp.int32)
    w_flat = W_E.reshape(D * V)
    out = _build(N, D, V)(x_flat, w_flat)
    return out.reshape(B, S, D)


# speedup vs baseline: 27.2602x; 27.0956x over previous
import functools

import jax
import jax.numpy as jnp
from jax import lax
from jax.experimental import pallas as pl
from jax.experimental.pallas import tpu as pltpu
from jax.experimental.pallas import tpu_sc as plsc

NUM_CORES = 2
NUM_SUBCORES = 16
NW = NUM_CORES * NUM_SUBCORES
CHUNK = 32


@functools.lru_cache(maxsize=None)
def _build(N, D, V):
    NI = N // NW
    NCH = NI // CHUNK

    mesh = plsc.VectorSubcoreMesh(
        core_axis_name="c", subcore_axis_name="s",
        num_cores=NUM_CORES, num_subcores=NUM_SUBCORES,
    )

    @functools.partial(
        pl.kernel,
        out_type=jax.ShapeDtypeStruct((N, D), jnp.float32),
        mesh=mesh,
        compiler_params=pltpu.CompilerParams(needs_layout_passes=False),
        scratch_types=[
            pltpu.VMEM((NI,), jnp.int32),
            pltpu.VMEM((CHUNK, D), jnp.float32),
            pltpu.VMEM((CHUNK, D), jnp.float32),
            pltpu.SemaphoreType.DMA,
            pltpu.SemaphoreType.DMA,
            pltpu.SemaphoreType.DMA,
            pltpu.SemaphoreType.DMA,
        ],
    )
    def run(x_hbm, wt_hbm, out_hbm, xl_v, rowsA, rowsB, sgA, sgB, soA, soB):
        c = lax.axis_index("c")
        s = lax.axis_index("s")
        wid = s * NUM_CORES + c
        i0 = wid * NI

        pltpu.sync_copy(x_hbm.at[pl.ds(i0, NI)], xl_v)

        def idx_slice(ch):
            return xl_v.at[pl.ds(ch * CHUNK, CHUNK)]

        def start_gather(ch, rows_ref, sem):
            pltpu.async_copy(wt_hbm.at[idx_slice(ch)], rows_ref, sem)

        def wait_gather(ch, rows_ref, sem):
            pltpu.make_async_copy(wt_hbm.at[idx_slice(ch)], rows_ref, sem).wait()

        def out_slice(ch):
            return out_hbm.at[pl.ds(i0 + ch * CHUNK, CHUNK), :]

        def start_out(rows_ref, ch, sem):
            pltpu.async_copy(rows_ref, out_slice(ch), sem)

        def wait_out(rows_ref, ch, sem):
            pltpu.make_async_copy(rows_ref, out_slice(ch), sem).wait()

        start_gather(0, rowsA, sgA)

        def body(h, _):
            a = 2 * h
            b = a + 1
            wait_gather(a, rowsA, sgA)

            @pl.when(h > 0)
            def _():
                wait_out(rowsB, b - 2, soB)
            start_gather(b, rowsB, sgB)
            start_out(rowsA, a, soA)
            wait_gather(b, rowsB, sgB)
            wait_out(rowsA, a, soA)

            @pl.when(h < NCH // 2 - 1)
            def _():
                start_gather(a + 2, rowsA, sgA)
            start_out(rowsB, b, soB)
            return 0

        lax.fori_loop(0, NCH // 2, body, 0)
        wait_out(rowsB, NCH - 1, soB)

    return run


def kernel(x, W_E):
    B, S = x.shape
    D, V = W_E.shape
    N = B * S
    x_flat = x.reshape(N).astype(jnp.int32)
    wt = W_E.T
    out = _build(N, D, V)(x_flat, wt)
    return out.reshape(B, S, D)


# speedup vs baseline: 27.7894x; 1.0194x over previous
import functools

import jax
import jax.numpy as jnp
from jax import lax
from jax.experimental import pallas as pl
from jax.experimental.pallas import tpu as pltpu
from jax.experimental.pallas import tpu_sc as plsc

NUM_CORES = 2
NUM_SUBCORES = 16
NW = NUM_CORES * NUM_SUBCORES
CHUNK = 32


@functools.lru_cache(maxsize=None)
def _build(N, D, V):
    NI = N // NW
    NCH = NI // CHUNK

    mesh = plsc.VectorSubcoreMesh(
        core_axis_name="c", subcore_axis_name="s",
        num_cores=NUM_CORES, num_subcores=NUM_SUBCORES,
    )

    @functools.partial(
        pl.kernel,
        out_type=jax.ShapeDtypeStruct((N, D), jnp.float32),
        mesh=mesh,
        compiler_params=pltpu.CompilerParams(needs_layout_passes=False),
        scratch_types=[
            pltpu.VMEM((NI,), jnp.int32),
            pltpu.VMEM((CHUNK, D), jnp.float32),
            pltpu.VMEM((CHUNK, D), jnp.float32),
            pltpu.VMEM((CHUNK, D), jnp.float32),
            pltpu.SemaphoreType.DMA,
            pltpu.SemaphoreType.DMA,
            pltpu.SemaphoreType.DMA,
            pltpu.SemaphoreType.DMA,
            pltpu.SemaphoreType.DMA,
            pltpu.SemaphoreType.DMA,
        ],
    )
    def run(x_hbm, wt_hbm, out_hbm, xl_v, r0, r1, r2,
            sg0, sg1, sg2, so0, so1, so2):
        c = lax.axis_index("c")
        s = lax.axis_index("s")
        wid = s * NUM_CORES + c
        i0 = wid * NI
        rows = [r0, r1, r2]
        sg = [sg0, sg1, sg2]
        so = [so0, so1, so2]

        pltpu.sync_copy(x_hbm.at[pl.ds(i0, NI)], xl_v)

        def idx_slice(ch):
            return xl_v.at[pl.ds(ch * CHUNK, CHUNK)]

        def start_gather(ch, rows_ref, sem):
            pltpu.async_copy(wt_hbm.at[idx_slice(ch)], rows_ref, sem)

        def wait_gather(ch, rows_ref, sem):
            pltpu.make_async_copy(wt_hbm.at[idx_slice(ch)], rows_ref, sem).wait()

        def out_slice(ch):
            return out_hbm.at[pl.ds(i0 + ch * CHUNK, CHUNK), :]

        def start_out(rows_ref, ch, sem):
            pltpu.async_copy(rows_ref, out_slice(ch), sem)

        def wait_out(rows_ref, ch, sem):
            pltpu.make_async_copy(rows_ref, out_slice(ch), sem).wait()

        for ch in range(NCH):
            b = ch % 3
            if ch >= 3:
                wait_out(rows[b], ch - 3, so[b])
            start_gather(ch, rows[b], sg[b])
            if ch >= 1:
                b2 = (ch - 1) % 3
                wait_gather(ch - 1, rows[b2], sg[b2])
                start_out(rows[b2], ch - 1, so[b2])
        bl = (NCH - 1) % 3
        wait_gather(NCH - 1, rows[bl], sg[bl])
        start_out(rows[bl], NCH - 1, so[bl])
        for ch in range(NCH - 3, NCH):
            b = ch % 3
            wait_out(rows[b], ch, so[b])

    return run


def kernel(x, W_E):
    B, S = x.shape
    D, V = W_E.shape
    N = B * S
    x_flat = x.reshape(N).astype(jnp.int32)
    wt = W_E.T
    out = _build(N, D, V)(x_flat, wt)
    return out.reshape(B, S, D)


# speedup vs baseline: 28.1716x; 1.0138x over previous
import functools

import jax
import jax.numpy as jnp
from jax import lax
from jax.experimental import pallas as pl
from jax.experimental.pallas import tpu as pltpu
from jax.experimental.pallas import tpu_sc as plsc

NUM_CORES = 2
NUM_SUBCORES = 16
NW = NUM_CORES * NUM_SUBCORES
CHUNK = 16
DEPTH = 5
LAG = 2


@functools.lru_cache(maxsize=None)
def _build(N, D, V):
    NI = N // NW
    NCH = NI // CHUNK

    mesh = plsc.VectorSubcoreMesh(
        core_axis_name="c", subcore_axis_name="s",
        num_cores=NUM_CORES, num_subcores=NUM_SUBCORES,
    )

    @functools.partial(
        pl.kernel,
        out_type=jax.ShapeDtypeStruct((N, D), jnp.float32),
        mesh=mesh,
        compiler_params=pltpu.CompilerParams(needs_layout_passes=False),
        scratch_types=(
            [pltpu.VMEM((NI,), jnp.int32)]
            + [pltpu.VMEM((CHUNK, D), jnp.float32) for _ in range(DEPTH)]
            + [pltpu.SemaphoreType.DMA for _ in range(2 * DEPTH)]
        ),
    )
    def run(x_hbm, wt_hbm, out_hbm, xl_v, *bufs_and_sems):
        rows = list(bufs_and_sems[:DEPTH])
        sg = list(bufs_and_sems[DEPTH:2 * DEPTH])
        so = list(bufs_and_sems[2 * DEPTH:])
        c = lax.axis_index("c")
        s = lax.axis_index("s")
        wid = s * NUM_CORES + c
        i0 = wid * NI

        pltpu.sync_copy(x_hbm.at[pl.ds(i0, NI)], xl_v)

        def idx_slice(ch):
            return xl_v.at[pl.ds(ch * CHUNK, CHUNK)]

        def start_gather(ch, rows_ref, sem):
            pltpu.async_copy(wt_hbm.at[idx_slice(ch)], rows_ref, sem)

        def wait_gather(ch, rows_ref, sem):
            pltpu.make_async_copy(wt_hbm.at[idx_slice(ch)], rows_ref, sem).wait()

        def out_slice(ch):
            return out_hbm.at[pl.ds(i0 + ch * CHUNK, CHUNK), :]

        def start_out(rows_ref, ch, sem):
            pltpu.async_copy(rows_ref, out_slice(ch), sem)

        def wait_out(rows_ref, ch, sem):
            pltpu.make_async_copy(rows_ref, out_slice(ch), sem).wait()

        for ch in range(NCH):
            b = ch % DEPTH
            if ch >= DEPTH:
                wait_out(rows[b], ch - DEPTH, so[b])
            start_gather(ch, rows[b], sg[b])
            if ch >= LAG:
                b2 = (ch - LAG) % DEPTH
                wait_gather(ch - LAG, rows[b2], sg[b2])
                start_out(rows[b2], ch - LAG, so[b2])
        for ch in range(NCH - LAG, NCH):
            b = ch % DEPTH
            wait_gather(ch, rows[b], sg[b])
            start_out(rows[b], ch, so[b])
        for ch in range(NCH - DEPTH, NCH):
            b = ch % DEPTH
            wait_out(rows[b], ch, so[b])

    return run


def kernel(x, W_E):
    B, S = x.shape
    D, V = W_E.shape
    N = B * S
    x_flat = x.reshape(N).astype(jnp.int32)
    wt = W_E.T
    out = _build(N, D, V)(x_flat, wt)
    return out.reshape(B, S, D)


# speedup vs baseline: 28.5409x; 1.0131x over previous
import functools

import jax
import jax.numpy as jnp
from jax import lax
from jax.experimental import pallas as pl
from jax.experimental.pallas import tpu as pltpu
from jax.experimental.pallas import tpu_sc as plsc

NUM_CORES = 2
NUM_SUBCORES = 16
NW = NUM_CORES * NUM_SUBCORES
CHUNK = 16
DEPTH = 6
LAG = 3


@functools.lru_cache(maxsize=None)
def _build(N, D, V):
    NI = N // NW
    NCH = NI // CHUNK

    mesh = plsc.VectorSubcoreMesh(
        core_axis_name="c", subcore_axis_name="s",
        num_cores=NUM_CORES, num_subcores=NUM_SUBCORES,
    )

    @functools.partial(
        pl.kernel,
        out_type=jax.ShapeDtypeStruct((N, D), jnp.float32),
        mesh=mesh,
        compiler_params=pltpu.CompilerParams(needs_layout_passes=False),
        scratch_types=(
            [pltpu.VMEM((NI,), jnp.int32)]
            + [pltpu.VMEM((CHUNK, D), jnp.float32) for _ in range(DEPTH)]
            + [pltpu.SemaphoreType.DMA for _ in range(2 * DEPTH)]
        ),
    )
    def run(x_hbm, wt_hbm, out_hbm, xl_v, *bufs_and_sems):
        rows = list(bufs_and_sems[:DEPTH])
        sg = list(bufs_and_sems[DEPTH:2 * DEPTH])
        so = list(bufs_and_sems[2 * DEPTH:])
        c = lax.axis_index("c")
        s = lax.axis_index("s")
        wid = s * NUM_CORES + c
        i0 = wid * NI

        pltpu.sync_copy(x_hbm.at[pl.ds(i0, NI)], xl_v)

        def idx_slice(ch):
            return xl_v.at[pl.ds(ch * CHUNK, CHUNK)]

        def start_gather(ch, rows_ref, sem):
            pltpu.async_copy(wt_hbm.at[idx_slice(ch)], rows_ref, sem)

        def wait_gather(ch, rows_ref, sem):
            pltpu.make_async_copy(wt_hbm.at[idx_slice(ch)], rows_ref, sem).wait()

        def out_slice(ch):
            return out_hbm.at[pl.ds(i0 + ch * CHUNK, CHUNK), :]

        def start_out(rows_ref, ch, sem):
            pltpu.async_copy(rows_ref, out_slice(ch), sem)

        def wait_out(rows_ref, ch, sem):
            pltpu.make_async_copy(rows_ref, out_slice(ch), sem).wait()

        for ch in range(NCH):
            b = ch % DEPTH
            if ch >= DEPTH:
                wait_out(rows[b], ch - DEPTH, so[b])
            start_gather(ch, rows[b], sg[b])
            if ch >= LAG:
                b2 = (ch - LAG) % DEPTH
                wait_gather(ch - LAG, rows[b2], sg[b2])
                start_out(rows[b2], ch - LAG, so[b2])
        for ch in range(NCH - LAG, NCH):
            b = ch % DEPTH
            wait_gather(ch, rows[b], sg[b])
            start_out(rows[b], ch, so[b])
        for ch in range(NCH - DEPTH, NCH):
            b = ch % DEPTH
            wait_out(rows[b], ch, so[b])

    return run


def kernel(x, W_E):
    B, S = x.shape
    D, V = W_E.shape
    N = B * S
    x_flat = x.reshape(N).astype(jnp.int32)
    wt = W_E.T
    out = _build(N, D, V)(x_flat, wt)
    return out.reshape(B, S, D)


# speedup vs baseline: 28.9268x; 1.0135x over previous
import functools

import jax
import jax.numpy as jnp
from jax import lax
from jax.experimental import pallas as pl
from jax.experimental.pallas import tpu as pltpu
from jax.experimental.pallas import tpu_sc as plsc

NUM_CORES = 2
NUM_SUBCORES = 16
NW = NUM_CORES * NUM_SUBCORES
CHUNK = 16
DEPTH = 7
LAG = 4


@functools.lru_cache(maxsize=None)
def _build(N, D, V):
    NI = N // NW
    NCH = NI // CHUNK

    mesh = plsc.VectorSubcoreMesh(
        core_axis_name="c", subcore_axis_name="s",
        num_cores=NUM_CORES, num_subcores=NUM_SUBCORES,
    )

    @functools.partial(
        pl.kernel,
        out_type=jax.ShapeDtypeStruct((N, D), jnp.float32),
        mesh=mesh,
        compiler_params=pltpu.CompilerParams(needs_layout_passes=False),
        scratch_types=(
            [pltpu.VMEM((NI,), jnp.int32)]
            + [pltpu.VMEM((CHUNK, D), jnp.float32) for _ in range(DEPTH)]
            + [pltpu.SemaphoreType.DMA for _ in range(2 * DEPTH)]
        ),
    )
    def run(x_hbm, wt_hbm, out_hbm, xl_v, *bufs_and_sems):
        rows = list(bufs_and_sems[:DEPTH])
        sg = list(bufs_and_sems[DEPTH:2 * DEPTH])
        so = list(bufs_and_sems[2 * DEPTH:])
        c = lax.axis_index("c")
        s = lax.axis_index("s")
        wid = s * NUM_CORES + c
        i0 = wid * NI

        pltpu.sync_copy(x_hbm.at[pl.ds(i0, NI)], xl_v)

        def idx_slice(ch):
            return xl_v.at[pl.ds(ch * CHUNK, CHUNK)]

        def start_gather(ch, rows_ref, sem):
            pltpu.async_copy(wt_hbm.at[idx_slice(ch)], rows_ref, sem)

        def wait_gather(ch, rows_ref, sem):
            pltpu.make_async_copy(wt_hbm.at[idx_slice(ch)], rows_ref, sem).wait()

        def out_slice(ch):
            return out_hbm.at[pl.ds(i0 + ch * CHUNK, CHUNK), :]

        def start_out(rows_ref, ch, sem):
            pltpu.async_copy(rows_ref, out_slice(ch), sem)

        def wait_out(rows_ref, ch, sem):
            pltpu.make_async_copy(rows_ref, out_slice(ch), sem).wait()

        for ch in range(NCH):
            b = ch % DEPTH
            if ch >= DEPTH:
                wait_out(rows[b], ch - DEPTH, so[b])
            start_gather(ch, rows[b], sg[b])
            if ch >= LAG:
                b2 = (ch - LAG) % DEPTH
                wait_gather(ch - LAG, rows[b2], sg[b2])
                start_out(rows[b2], ch - LAG, so[b2])
        for ch in range(NCH - LAG, NCH):
            b = ch % DEPTH
            wait_gather(ch, rows[b], sg[b])
            start_out(rows[b], ch, so[b])
        for ch in range(NCH - DEPTH, NCH):
            b = ch % DEPTH
            wait_out(rows[b], ch, so[b])

    return run


def kernel(x, W_E):
    B, S = x.shape
    D, V = W_E.shape
    N = B * S
    x_flat = x.reshape(N).astype(jnp.int32)
    wt = W_E.T
    out = _build(N, D, V)(x_flat, wt)
    return out.reshape(B, S, D)
